# Initial kernel scaffold; baseline (speedup 1.0000x reference)
import jax
import jax.numpy as jnp
from jax.experimental import pallas as pl


def kernel(t, l, u, emb_t, emb_l, emb_u):
    raise NotImplementedError("write your pallas kernel here")


# speedup vs baseline: 4.5270x; 4.5270x over previous
import functools

import jax
import jax.numpy as jnp
from jax import lax
from jax.experimental import pallas as pl
from jax.experimental.pallas import tpu as pltpu
from jax.experimental.pallas import tpu_sc as plsc

_LANES = 16
_C = 128


@functools.lru_cache(maxsize=None)
def _build_sc_call(N, D):
    info = plsc.get_sparse_core_info()
    num_workers = info.num_cores * info.num_subcores
    K = 512
    KC = K // _C
    per_w = N // num_workers
    num_batches = per_w // K
    assert per_w % K == 0 and N % num_workers == 0

    mesh = plsc.VectorSubcoreMesh(core_axis_name="c", subcore_axis_name="s")

    @functools.partial(
        pl.kernel,
        mesh=mesh,
        compiler_params=pltpu.CompilerParams(use_tc_tiling_on_sc=False),
        out_type=jax.ShapeDtypeStruct((N, D), jnp.float32),
        scratch_types=[
            pltpu.VMEM((KC, _C), jnp.int32),
            pltpu.VMEM((KC, _C), jnp.int32),
            pltpu.VMEM((KC, _C), jnp.int32),
            pltpu.VMEM((K, D), jnp.float32),
            pltpu.VMEM((K, D), jnp.float32),
            pltpu.VMEM((K, D), jnp.float32),
            pltpu.SemaphoreType.DMA,
            pltpu.SemaphoreType.DMA,
        ],
    )
    def sc_fn(t_hbm, l_hbm, u_hbm, et_hbm, el_hbm, eu_hbm, out_hbm,
              it_v, il_v, iu_v, rt_v, rl_v, ru_v, isem, gsem):
        wid = lax.axis_index("s") * info.num_cores + lax.axis_index("c")
        idx_row0 = wid * (per_w // _C)
        out_row0 = wid * per_w

        def batch(bi, carry):
            irow = idx_row0 + bi * KC
            ct = pltpu.async_copy(t_hbm.at[pl.ds(irow, KC)], it_v, isem)
            cl = pltpu.async_copy(l_hbm.at[pl.ds(irow, KC)], il_v, isem)
            cu = pltpu.async_copy(u_hbm.at[pl.ds(irow, KC)], iu_v, isem)
            ct.wait()
            cl.wait()
            cu.wait()
            copies = []
            for j in range(KC):
                dst = pl.ds(j * _C, _C)
                copies.append(pltpu.async_copy(et_hbm.at[it_v.at[j]], rt_v.at[dst], gsem))
                copies.append(pltpu.async_copy(el_hbm.at[il_v.at[j]], rl_v.at[dst], gsem))
                copies.append(pltpu.async_copy(eu_hbm.at[iu_v.at[j]], ru_v.at[dst], gsem))
            for c in copies:
                c.wait()

            def add_row(i, acc):
                for h in range(D // _LANES):
                    sl = pl.ds(h * _LANES, _LANES)
                    rt_v[i, sl] = rt_v[i, sl] + rl_v[i, sl] + ru_v[i, sl]
                return acc

            lax.fori_loop(0, K, add_row, 0)
            pltpu.sync_copy(rt_v, out_hbm.at[pl.ds(out_row0 + bi * K, K)])
            return carry

        lax.fori_loop(0, num_batches, batch, 0)

    return sc_fn


def kernel(t, l, u, emb_t, emb_l, emb_u):
    B, S = t.shape
    N = B * S
    D = emb_t.shape[1]
    ti = t.reshape(N // _C, _C).astype(jnp.int32)
    li = l.reshape(N // _C, _C).astype(jnp.int32)
    ui = u.reshape(N // _C, _C).astype(jnp.int32)
    out = _build_sc_call(N, D)(ti, li, ui, emb_t, emb_l, emb_u)
    return out.reshape(B, S, D)


# speedup vs baseline: 4.9809x; 1.1003x over previous
import functools

import jax
import jax.numpy as jnp
from jax import lax
from jax.experimental import pallas as pl
from jax.experimental.pallas import tpu as pltpu
from jax.experimental.pallas import tpu_sc as plsc

_LANES = 16
_C = 128


@functools.lru_cache(maxsize=None)
def _build_sc_call(N, D):
    info = plsc.get_sparse_core_info()
    num_workers = info.num_cores * info.num_subcores
    K = 512
    KC = K // _C
    per_w = N // num_workers
    num_batches = per_w // K
    assert per_w % K == 0 and N % num_workers == 0

    mesh = plsc.VectorSubcoreMesh(core_axis_name="c", subcore_axis_name="s")

    @functools.partial(
        pl.kernel,
        mesh=mesh,
        compiler_params=pltpu.CompilerParams(use_tc_tiling_on_sc=False),
        out_type=jax.ShapeDtypeStruct((N, D), jnp.float32),
        scratch_types=[
            pltpu.VMEM((KC, _C), jnp.int32),
            pltpu.VMEM((KC, _C), jnp.int32),
            pltpu.VMEM((KC, _C), jnp.int32),
            pltpu.VMEM((K, D), jnp.float32),
            pltpu.VMEM((K, D), jnp.float32),
            pltpu.VMEM((K, D), jnp.float32),
            pltpu.SemaphoreType.DMA,
            pltpu.SemaphoreType.DMA,
        ],
    )
    def sc_fn(t_hbm, l_hbm, u_hbm, et_hbm, el_hbm, eu_hbm, out_hbm,
              it_v, il_v, iu_v, rt_v, rl_v, ru_v, isem, gsem):
        wid = lax.axis_index("s") * info.num_cores + lax.axis_index("c")
        idx_row0 = wid * (per_w // _C)
        out_row0 = wid * per_w

        def batch(bi, carry):
            irow = idx_row0 + bi * KC
            ct = pltpu.async_copy(t_hbm.at[pl.ds(irow, KC)], it_v, isem)
            cl = pltpu.async_copy(l_hbm.at[pl.ds(irow, KC)], il_v, isem)
            cu = pltpu.async_copy(u_hbm.at[pl.ds(irow, KC)], iu_v, isem)
            ct.wait()
            cl.wait()
            cu.wait()
            copies = []
            for j in range(KC):
                dst = pl.ds(j * _C, _C)
                copies.append(pltpu.async_copy(et_hbm.at[it_v.at[j]], rt_v.at[dst], gsem))
            for c in copies:
                c.wait()
            copies = []
            for j in range(KC):
                dst = pl.ds(j * _C, _C)
                copies.append(pltpu.async_copy(el_hbm.at[il_v.at[j]], rt_v.at[dst], gsem, add=True))
                copies.append(pltpu.async_copy(eu_hbm.at[iu_v.at[j]], rt_v.at[dst], gsem, add=True))
            for c in copies:
                c.wait()
            pltpu.sync_copy(rt_v, out_hbm.at[pl.ds(out_row0 + bi * K, K)])
            return carry

        lax.fori_loop(0, num_batches, batch, 0)

    return sc_fn


def kernel(t, l, u, emb_t, emb_l, emb_u):
    B, S = t.shape
    N = B * S
    D = emb_t.shape[1]
    ti = t.reshape(N // _C, _C).astype(jnp.int32)
    li = l.reshape(N // _C, _C).astype(jnp.int32)
    ui = u.reshape(N // _C, _C).astype(jnp.int32)
    out = _build_sc_call(N, D)(ti, li, ui, emb_t, emb_l, emb_u)
    return out.reshape(B, S, D)


# speedup vs baseline: 5.1478x; 1.0335x over previous
import functools

import jax
import jax.numpy as jnp
from jax import lax
from jax.experimental import pallas as pl
from jax.experimental.pallas import tpu as pltpu
from jax.experimental.pallas import tpu_sc as plsc

_C = 128
_K = 256
_S = 4


@functools.lru_cache(maxsize=None)
def _build_sc_call(N, D):
    info = plsc.get_sparse_core_info()
    num_workers = info.num_cores * info.num_subcores
    KC = _K // _C
    per_w = N // num_workers
    idx_rows = per_w // _C
    nb = per_w // _K
    assert N % num_workers == 0 and per_w % _K == 0 and nb % _S == 0 and nb > 2 * _S

    mesh = plsc.VectorSubcoreMesh(core_axis_name="c", subcore_axis_name="s")

    @functools.partial(
        pl.kernel,
        mesh=mesh,
        compiler_params=pltpu.CompilerParams(use_tc_tiling_on_sc=False),
        out_type=jax.ShapeDtypeStruct((N, D), jnp.float32),
        scratch_types=(
            [pltpu.VMEM((idx_rows, _C), jnp.int32) for _ in range(3)]
            + [pltpu.VMEM((_K, D), jnp.float32) for _ in range(_S)]
            + [pltpu.SemaphoreType.DMA for _ in range(3 * _S + 1)]
        ),
    )
    def sc_fn(t_hbm, l_hbm, u_hbm, et_hbm, el_hbm, eu_hbm, out_hbm, *refs):
        it_v, il_v, iu_v = refs[0:3]
        acc = refs[3:3 + _S]
        tsem = refs[3 + _S:3 + 2 * _S]
        asem = refs[3 + 2 * _S:3 + 3 * _S]
        osem = refs[3 + 3 * _S:3 + 4 * _S]
        isem = refs[3 + 4 * _S]

        wid = lax.axis_index("s") * info.num_cores + lax.axis_index("c")
        idx_row0 = wid * idx_rows
        out_row0 = wid * per_w

        ci = pltpu.async_copy(t_hbm.at[pl.ds(idx_row0, idx_rows)], it_v, isem)
        cl = pltpu.async_copy(l_hbm.at[pl.ds(idx_row0, idx_rows)], il_v, isem)
        cu = pltpu.async_copy(u_hbm.at[pl.ds(idx_row0, idx_rows)], iu_v, isem)
        ci.wait()
        cl.wait()
        cu.wait()

        def t_gather(bi, s, issue):
            for k in range(KC):
                d = pltpu.make_async_copy(
                    et_hbm.at[it_v.at[bi * KC + k]], acc[s].at[pl.ds(k * _C, _C)], tsem[s])
                d.start() if issue else d.wait()

        def add_gathers(bi, s, issue):
            for k in range(KC):
                dst = acc[s].at[pl.ds(k * _C, _C)]
                dl = pltpu.make_async_copy(el_hbm.at[il_v.at[bi * KC + k]], dst, asem[s])
                du = pltpu.make_async_copy(eu_hbm.at[iu_v.at[bi * KC + k]], dst, asem[s])
                if issue:
                    dl.start(add=True)
                    du.start(add=True)
                else:
                    dl.wait()
                    du.wait()

        def out_copy(bi, s, issue):
            d = pltpu.make_async_copy(acc[s], out_hbm.at[pl.ds(out_row0 + bi * _K, _K)], osem[s])
            d.start() if issue else d.wait()

        def slot_b(i, s):
            t_gather(i - 1, s, False)
            add_gathers(i - 1, s, True)

        def slot_c(i, s):
            add_gathers(i - 2, s, False)
            out_copy(i - 2, s, True)

        for i in range(_S):
            t_gather(i, i % _S, True)
            if i >= 1:
                slot_b(i, (i - 1) % _S)
            if i >= 2:
                slot_c(i, (i - 2) % _S)

        def group(g, carry):
            for s in range(_S):
                i = g * _S + s
                out_copy(i - _S, s, False)
                t_gather(i, s, True)
                slot_b(i, (s - 1) % _S)
                slot_c(i, (s - 2) % _S)
            return carry

        lax.fori_loop(1, nb // _S, group, 0)

        slot_b(nb, (nb - 1) % _S)
        slot_c(nb, (nb - 2) % _S)
        slot_c(nb + 1, (nb - 1) % _S)
        for s in range(_S):
            out_copy(nb - _S + s, s, False)

    return sc_fn


def kernel(t, l, u, emb_t, emb_l, emb_u):
    B, S = t.shape
    N = B * S
    D = emb_t.shape[1]
    ti = t.reshape(N // _C, _C).astype(jnp.int32)
    li = l.reshape(N // _C, _C).astype(jnp.int32)
    ui = u.reshape(N // _C, _C).astype(jnp.int32)
    out = _build_sc_call(N, D)(ti, li, ui, emb_t, emb_l, emb_u)
    return out.reshape(B, S, D)


# speedup vs baseline: 5.1499x; 1.0004x over previous
import functools

import jax
import jax.numpy as jnp
from jax import lax
from jax.experimental import pallas as pl
from jax.experimental.pallas import tpu as pltpu
from jax.experimental.pallas import tpu_sc as plsc

_S_SETS = 4


@functools.lru_cache(maxsize=None)
def _build_sc_call(B, S, D):
    info = plsc.get_sparse_core_info()
    num_workers = info.num_cores * info.num_subcores
    per_w = B // num_workers
    assert B % num_workers == 0 and per_w % _S_SETS == 0 and per_w > 2 * _S_SETS
    chunks = []
    off = 0
    while off < S:
        n = min(128, S - off)
        chunks.append((off, n))
        off += n
    assert all(o % 8 == 0 for o, _ in chunks)

    mesh = plsc.VectorSubcoreMesh(core_axis_name="c", subcore_axis_name="s")

    @functools.partial(
        pl.kernel,
        mesh=mesh,
        compiler_params=pltpu.CompilerParams(use_tc_tiling_on_sc=False),
        out_type=jax.ShapeDtypeStruct((B, S, D), jnp.float32),
        scratch_types=(
            [pltpu.VMEM((per_w, S), jnp.int32) for _ in range(3)]
            + [pltpu.VMEM((S, D), jnp.float32) for _ in range(_S_SETS)]
            + [pltpu.SemaphoreType.DMA for _ in range(3 * _S_SETS + 1)]
        ),
    )
    def sc_fn(t_hbm, l_hbm, u_hbm, et_hbm, el_hbm, eu_hbm, out_hbm, *refs):
        it_v, il_v, iu_v = refs[0:3]
        acc = refs[3:3 + _S_SETS]
        tsem = refs[3 + _S_SETS:3 + 2 * _S_SETS]
        asem = refs[3 + 2 * _S_SETS:3 + 3 * _S_SETS]
        osem = refs[3 + 3 * _S_SETS:3 + 4 * _S_SETS]
        isem = refs[3 + 4 * _S_SETS]

        wid = lax.axis_index("s") * info.num_cores + lax.axis_index("c")
        b0 = wid * per_w

        ct = pltpu.async_copy(t_hbm.at[pl.ds(b0, per_w)], it_v, isem)
        cl = pltpu.async_copy(l_hbm.at[pl.ds(b0, per_w)], il_v, isem)
        cu = pltpu.async_copy(u_hbm.at[pl.ds(b0, per_w)], iu_v, isem)
        ct.wait()
        cl.wait()
        cu.wait()

        def t_gather(bi, s, issue):
            for off, n in chunks:
                d = pltpu.make_async_copy(
                    et_hbm.at[it_v.at[bi, pl.ds(off, n)]],
                    acc[s].at[pl.ds(off, n)], tsem[s])
                d.start() if issue else d.wait()

        def add_gathers(bi, s, issue):
            for off, n in chunks:
                dst = acc[s].at[pl.ds(off, n)]
                dl = pltpu.make_async_copy(el_hbm.at[il_v.at[bi, pl.ds(off, n)]], dst, asem[s])
                du = pltpu.make_async_copy(eu_hbm.at[iu_v.at[bi, pl.ds(off, n)]], dst, asem[s])
                if issue:
                    dl.start(add=True)
                    du.start(add=True)
                else:
                    dl.wait()
                    du.wait()

        def out_copy(bi, s, issue):
            d = pltpu.make_async_copy(acc[s], out_hbm.at[b0 + bi], osem[s])
            d.start() if issue else d.wait()

        def slot_b(i, s):
            t_gather(i - 1, s, False)
            add_gathers(i - 1, s, True)

        def slot_c(i, s):
            add_gathers(i - 2, s, False)
            out_copy(i - 2, s, True)

        for i in range(_S_SETS):
            t_gather(i, i % _S_SETS, True)
            if i >= 1:
                slot_b(i, (i - 1) % _S_SETS)
            if i >= 2:
                slot_c(i, (i - 2) % _S_SETS)

        def group(g, carry):
            for s in range(_S_SETS):
                i = g * _S_SETS + s
                out_copy(i - _S_SETS, s, False)
                t_gather(i, s, True)
                slot_b(i, (s - 1) % _S_SETS)
                slot_c(i, (s - 2) % _S_SETS)
            return carry

        lax.fori_loop(1, per_w // _S_SETS, group, 0)

        slot_b(per_w, (per_w - 1) % _S_SETS)
        slot_c(per_w, (per_w - 2) % _S_SETS)
        slot_c(per_w + 1, (per_w - 1) % _S_SETS)
        for s in range(_S_SETS):
            out_copy(per_w - _S_SETS + s, s, False)

    return sc_fn


def kernel(t, l, u, emb_t, emb_l, emb_u):
    B, S = t.shape
    D = emb_t.shape[1]
    return _build_sc_call(B, S, D)(
        t.astype(jnp.int32), l.astype(jnp.int32), u.astype(jnp.int32),
        emb_t, emb_l, emb_u)
